# baseline (speedup 1.0000x reference)
import functools

import jax
import jax.numpy as jnp
from jax import lax
from jax.experimental import pallas as pl
from jax.experimental.pallas import tpu as pltpu
from jax.experimental.pallas import tpu_sc as plsc

N = 10000
E = 320000
H = 4
C = 32
HID = 128
FIN = 128
ED = 16
HP = 16
K = 64
NW = 32
TPW = 162
EPAD = TPW * NW * K
AEROWS = EPAD // 8
NP = 10240
TROWS = NP // 16
ZR = K


def _prep_nodes(x2d, W, As, Ad):
    BN = 1000

    def body(x_ref, w_ref, as_ref, ad_ref, xw_ref, als_ref, ald_ref):
        xw = jnp.dot(x_ref[...], w_ref[...], preferred_element_type=jnp.float32)
        xw_ref[...] = xw
        als_ref[...] = jnp.dot(xw, as_ref[...], preferred_element_type=jnp.float32)
        ald_ref[...] = jnp.dot(xw, ad_ref[...], preferred_element_type=jnp.float32)

    return pl.pallas_call(
        body,
        grid=(N // BN,),
        in_specs=[
            pl.BlockSpec((BN, FIN), lambda i: (i, 0)),
            pl.BlockSpec((FIN, HID), lambda i: (0, 0)),
            pl.BlockSpec((HID, HP), lambda i: (0, 0)),
            pl.BlockSpec((HID, HP), lambda i: (0, 0)),
        ],
        out_specs=[
            pl.BlockSpec((BN, HID), lambda i: (i, 0)),
            pl.BlockSpec((BN, HP), lambda i: (i, 0)),
            pl.BlockSpec((BN, HP), lambda i: (i, 0)),
        ],
        out_shape=[
            jax.ShapeDtypeStruct((N, HID), jnp.float32),
            jax.ShapeDtypeStruct((N, HP), jnp.float32),
            jax.ShapeDtypeStruct((N, HP), jnp.float32),
        ],
    )(x2d, W, As, Ad)


def _prep_edges(ea_pack, BD):
    BE = 4000

    def body(ea_ref, bd_ref, ae_ref):
        ae_ref[...] = jnp.dot(ea_ref[...], bd_ref[...],
                              preferred_element_type=jnp.float32)

    return pl.pallas_call(
        body,
        grid=(E // 8 // BE,),
        in_specs=[
            pl.BlockSpec((BE, 128), lambda i: (i, 0)),
            pl.BlockSpec((128, 128), lambda i: (0, 0)),
        ],
        out_specs=pl.BlockSpec((BE, 128), lambda i: (i, 0)),
        out_shape=jax.ShapeDtypeStruct((AEROWS, 128), jnp.float32),
    )(ea_pack, BD)


def _sc_gat(src, dst, ae, asrc, adst, xw):
    mesh = plsc.VectorSubcoreMesh(core_axis_name="c", subcore_axis_name="s")

    @functools.partial(
        pl.kernel,
        out_type=[
            jax.ShapeDtypeStruct((2, NP, HID), jnp.float32),
            jax.ShapeDtypeStruct((2, NP, HP), jnp.float32),
        ],
        mesh=mesh,
        compiler_params=pltpu.CompilerParams(use_tc_tiling_on_sc=False),
        scratch_types=[
            pltpu.VMEM_SHARED((NP, HID), jnp.float32),
            pltpu.VMEM_SHARED((NP, HP), jnp.float32),
            pltpu.VMEM((3, K), jnp.int32),
            pltpu.VMEM((3, K), jnp.int32),
            pltpu.VMEM((3, K, HP), jnp.float32),
            pltpu.VMEM((3, K, HP), jnp.float32),
            pltpu.VMEM((3, K // 8, HID), jnp.float32),
            pltpu.VMEM((3, K, HP), jnp.float32),
            pltpu.VMEM((3, K, HID), jnp.float32),
        ] + [pltpu.SemaphoreType.DMA] * 9,
    )
    def k(src_hbm, dst_hbm, ae_hbm, asrc_hbm, adst_hbm, xw_hbm,
          out_hbm, den_hbm,
          acc_s, den_s, idx_s2, idx_d2, asrc_v, adst_v, ae_v, w_v, rows_v,
          sl0, sl1, sl2, sg0, sg1, sg2, ss0, ss1, ss2):
        sem_lin = (sl0, sl1, sl2)
        sem_gat = (sg0, sg1, sg2)
        sem_sca = (ss0, ss1, ss2)
        cid = lax.axis_index("c")
        sid = lax.axis_index("s")
        wid = sid * 2 + cid
        zv = jnp.zeros((16,), jnp.float32)

        def issue_lin(b, u):
            ch = u * NW + wid
            base = ch * K
            return [
                pltpu.async_copy(src_hbm.at[pl.ds(base, K)], idx_s2.at[b],
                                 sem_lin[b]),
                pltpu.async_copy(dst_hbm.at[pl.ds(base, K)], idx_d2.at[b],
                                 sem_lin[b]),
                pltpu.async_copy(ae_hbm.at[pl.ds(ch * (K // 8), K // 8)],
                                 ae_v.at[b], sem_lin[b]),
            ]

        def issue_gat(b):
            return [
                pltpu.async_copy(asrc_hbm.at[idx_s2.at[b]], asrc_v.at[b],
                                 sem_gat[b]),
                pltpu.async_copy(adst_hbm.at[idx_d2.at[b]], adst_v.at[b],
                                 sem_gat[b]),
                pltpu.async_copy(xw_hbm.at[idx_s2.at[b]], rows_v.at[b],
                                 sem_gat[b]),
            ]

        def issue_sca(b):
            return [
                pltpu.async_copy(w_v.at[b], den_s.at[idx_d2.at[b]],
                                 sem_sca[b], add=True),
                pltpu.async_copy(rows_v.at[b], acc_s.at[idx_d2.at[b]],
                                 sem_sca[b], add=True),
            ]

        def zbody(r, _):
            for j in range(HID // 16):
                rows_v[0, r, pl.ds(j * 16, 16)] = zv
            w_v[0, r, :] = zv
            return 0

        lax.fori_loop(0, K, zbody, 0)
        row0 = sid * TROWS
        for t in range(TROWS // ZR):
            pltpu.sync_copy(rows_v.at[0], acc_s.at[pl.ds(row0 + t * ZR, ZR)])
            pltpu.sync_copy(w_v.at[0], den_s.at[pl.ds(row0 + t * ZR, ZR)])
        plsc.subcore_barrier()

        gdn = lax.GatherDimensionNumbers(
            offset_dims=(), collapsed_slice_dims=(0,), start_index_map=(0,))

        def compute(b):
            def wbody(g, _):
                for p in range(8):
                    i = g * 8 + p
                    a = (asrc_v[b, i, :] + adst_v[b, i, :]
                         + ae_v[b, g, pl.ds(p * 16, 16)])
                    a = jnp.where(a >= 0.0, a, 0.2 * a)
                    w_v[b, i, :] = jnp.exp(a)
                return 0

            lax.fori_loop(0, K // 8, wbody, 0)

            def mbody(e, _):
                wrow = w_v[b, e, :]
                for j in range(HID // 16):
                    hidx = jnp.full((16, 1), j // 2, jnp.int32)
                    wb = lax.gather(
                        wrow, hidx, gdn, (1,),
                        mode=lax.GatherScatterMode.PROMISE_IN_BOUNDS)
                    sl = pl.ds(j * 16, 16)
                    rows_v[b, e, sl] = rows_v[b, e, sl] * wb
                return 0

            lax.fori_loop(0, K, mbody, 0)

        def triple(v, _):
            u0 = 3 * v
            l0 = issue_lin(0, u0)
            l1 = issue_lin(1, u0 + 1)
            l2 = issue_lin(2, u0 + 2)
            for d in l0:
                d.wait()
            g0 = issue_gat(0)
            for d in l1:
                d.wait()
            g1 = issue_gat(1)
            for d in g0:
                d.wait()
            compute(0)
            s0 = issue_sca(0)
            for d in l2:
                d.wait()
            g2 = issue_gat(2)
            for d in g1:
                d.wait()
            compute(1)
            s1 = issue_sca(1)
            for d in g2:
                d.wait()
            compute(2)
            s2 = issue_sca(2)
            for d in s0 + s1 + s2:
                d.wait()
            return 0

        lax.fori_loop(0, TPW // 3, triple, 0)
        plsc.subcore_barrier()

        for t in range(TROWS // ZR):
            r = row0 + t * ZR
            pltpu.sync_copy(acc_s.at[pl.ds(r, ZR)], rows_v.at[0])
            pltpu.sync_copy(rows_v.at[0], out_hbm.at[cid, pl.ds(r, ZR)])
            pltpu.sync_copy(den_s.at[pl.ds(r, ZR)], w_v.at[0])
            pltpu.sync_copy(w_v.at[0], den_hbm.at[cid, pl.ds(r, ZR)])

    return k(src, dst, ae, asrc, adst, xw)


def _post(outp, denp, bgat2, W_ih, bias2, E4, gamma2, beta2):
    BN = 1000

    def body(op_ref, dp_ref, bg_ref, wih_ref, b2_ref, e4_ref, ga_ref, be_ref,
             hout_ref, h_ref, c_ref):
        s = op_ref[0] + op_ref[1]
        den = dp_ref[0] + dp_ref[1]
        denf = jnp.dot(den, e4_ref[...], preferred_element_type=jnp.float32)
        sf = s / (denf + 1e-16) + bg_ref[...]
        gates = lax.dot_general(sf, wih_ref[...],
                                (((1,), (1,)), ((), ())),
                                preferred_element_type=jnp.float32) + b2_ref[...]
        ig = jax.nn.sigmoid(gates[:, 0:HID])
        gg = jnp.tanh(gates[:, 2 * HID:3 * HID])
        og = jax.nn.sigmoid(gates[:, 3 * HID:4 * HID])
        c = ig * gg
        h = og * jnp.tanh(c)
        mu = jnp.mean(h, axis=1, keepdims=True)
        var = jnp.mean((h - mu) * (h - mu), axis=1, keepdims=True)
        hn = (h - mu) / jnp.sqrt(var + 1e-5) * ga_ref[...] + be_ref[...]
        hout_ref[...] = hn
        h_ref[...] = h
        c_ref[...] = c

    return pl.pallas_call(
        body,
        grid=(N // BN,),
        in_specs=[
            pl.BlockSpec((2, BN, HID), lambda i: (0, i, 0)),
            pl.BlockSpec((2, BN, HP), lambda i: (0, i, 0)),
            pl.BlockSpec((1, HID), lambda i: (0, 0)),
            pl.BlockSpec((4 * HID, HID), lambda i: (0, 0)),
            pl.BlockSpec((1, 4 * HID), lambda i: (0, 0)),
            pl.BlockSpec((HP, HID), lambda i: (0, 0)),
            pl.BlockSpec((1, HID), lambda i: (0, 0)),
            pl.BlockSpec((1, HID), lambda i: (0, 0)),
        ],
        out_specs=[
            pl.BlockSpec((BN, HID), lambda i: (i, 0)),
            pl.BlockSpec((BN, HID), lambda i: (i, 0)),
            pl.BlockSpec((BN, HID), lambda i: (i, 0)),
        ],
        out_shape=[
            jax.ShapeDtypeStruct((N, HID), jnp.float32),
            jax.ShapeDtypeStruct((N, HID), jnp.float32),
            jax.ShapeDtypeStruct((N, HID), jnp.float32),
        ],
    )(outp, denp, bgat2, W_ih, bias2, E4, gamma2, beta2)


def kernel(x, edge_index, edge_attr, W, att_src, att_dst, att_edge, W_edge,
           b_gat, W_ih, W_hh, b_ih, b_hh, gamma, beta):
    x2d = x.reshape(N, FIN)
    pad_e = EPAD - E
    src = jnp.concatenate([edge_index[0], jnp.zeros((pad_e,), jnp.int32)])
    dst = jnp.concatenate([edge_index[1], jnp.full((pad_e,), N, jnp.int32)])

    hmask = (jnp.arange(HP)[None, :] == jnp.arange(H)[:, None]).astype(jnp.float32)
    As = (att_src[:, :, None] * hmask[:, None, :]).reshape(FIN, HP)
    Ad = (att_dst[:, :, None] * hmask[:, None, :]).reshape(FIN, HP)
    Vp = ((W_edge.reshape(ED, H, C) * att_edge[None]).sum(-1) @ hmask)
    BD = jnp.kron(jnp.eye(8, dtype=jnp.float32), Vp)
    E4 = (jnp.arange(HP)[:, None] == (jnp.arange(HID)[None, :] // C)).astype(jnp.float32)
    bias2 = (b_ih + b_hh).reshape(1, 4 * HID)
    bgat2 = b_gat.reshape(1, HID)
    gamma2 = gamma.reshape(1, HID)
    beta2 = beta.reshape(1, HID)

    xw, asrc, adst = _prep_nodes(x2d, W, As, Ad)
    zpad = jnp.zeros((NP - N, HP), jnp.float32)
    asrc = jnp.concatenate([asrc, zpad])
    adst = jnp.concatenate([adst, zpad])
    ae = _prep_edges(edge_attr.reshape(E // 8, 128), BD)
    outp, denp = _sc_gat(src, dst, ae, asrc, adst, xw)
    hout, h, c = _post(outp, denp, bgat2, W_ih, bias2, E4, gamma2, beta2)
    return (hout.reshape(1, N, HID), h.reshape(1, N, HID), c.reshape(1, N, HID))


# speedup vs baseline: 1.1472x; 1.1472x over previous
import functools

import jax
import jax.numpy as jnp
from jax import lax
from jax.experimental import pallas as pl
from jax.experimental.pallas import tpu as pltpu
from jax.experimental.pallas import tpu_sc as plsc

N = 10000
E = 320000
H = 4
C = 32
HID = 128
FIN = 128
ED = 16
HP = 16
K = 128
NW = 32
TPW = 80
EPAD = TPW * NW * K
AEROWS = EPAD // 8
NP = 10240
TROWS = NP // 16
ZR = K


def _prep_nodes(x2d, W, As, Ad):
    BN = 1000

    def body(x_ref, w_ref, as_ref, ad_ref, xw_ref, als_ref, ald_ref):
        xw = jnp.dot(x_ref[...], w_ref[...], preferred_element_type=jnp.float32)
        xw_ref[...] = xw
        als_ref[...] = jnp.dot(xw, as_ref[...], preferred_element_type=jnp.float32)
        ald_ref[...] = jnp.dot(xw, ad_ref[...], preferred_element_type=jnp.float32)

    return pl.pallas_call(
        body,
        grid=(N // BN,),
        in_specs=[
            pl.BlockSpec((BN, FIN), lambda i: (i, 0)),
            pl.BlockSpec((FIN, HID), lambda i: (0, 0)),
            pl.BlockSpec((HID, HP), lambda i: (0, 0)),
            pl.BlockSpec((HID, HP), lambda i: (0, 0)),
        ],
        out_specs=[
            pl.BlockSpec((BN, HID), lambda i: (i, 0)),
            pl.BlockSpec((BN, HP), lambda i: (i, 0)),
            pl.BlockSpec((BN, HP), lambda i: (i, 0)),
        ],
        out_shape=[
            jax.ShapeDtypeStruct((N, HID), jnp.float32),
            jax.ShapeDtypeStruct((N, HP), jnp.float32),
            jax.ShapeDtypeStruct((N, HP), jnp.float32),
        ],
    )(x2d, W, As, Ad)


def _prep_edges(ea_pack, BD):
    BE = 4000

    def body(ea_ref, bd_ref, ae_ref):
        ae_ref[...] = jnp.dot(ea_ref[...], bd_ref[...],
                              preferred_element_type=jnp.float32)

    return pl.pallas_call(
        body,
        grid=(E // 8 // BE,),
        in_specs=[
            pl.BlockSpec((BE, 128), lambda i: (i, 0)),
            pl.BlockSpec((128, 128), lambda i: (0, 0)),
        ],
        out_specs=pl.BlockSpec((BE, 128), lambda i: (i, 0)),
        out_shape=jax.ShapeDtypeStruct((AEROWS, 128), jnp.float32),
    )(ea_pack, BD)


def _sc_gat(rec, ae, asrc, adst, xw):
    mesh = plsc.VectorSubcoreMesh(core_axis_name="c", subcore_axis_name="s")

    @functools.partial(
        pl.kernel,
        out_type=[
            jax.ShapeDtypeStruct((2, NP, HID), jnp.float32),
            jax.ShapeDtypeStruct((2, NP, HP), jnp.float32),
        ],
        mesh=mesh,
        compiler_params=pltpu.CompilerParams(use_tc_tiling_on_sc=False),
        scratch_types=[
            pltpu.VMEM_SHARED((NP, HID), jnp.float32),
            pltpu.VMEM_SHARED((NP, HP), jnp.float32),
            pltpu.VMEM((2, K), jnp.int32),
            pltpu.VMEM((K // 8, 128), jnp.float32),
            pltpu.VMEM((K, HP), jnp.float32),
            pltpu.VMEM((K, HP), jnp.float32),
            pltpu.VMEM((K, HP), jnp.float32),
            pltpu.VMEM((K, HID), jnp.float32),
            pltpu.SemaphoreType.DMA,
            pltpu.SemaphoreType.DMA,
            pltpu.SemaphoreType.DMA,
        ],
    )
    def k(rec_hbm, ae_hbm, asrc_hbm, adst_hbm, xw_hbm,
          out_hbm, den_hbm,
          acc_s, den_s, rec_v, ae_v, asrc_v, adst_v, w_v, rows_v,
          sem_a, sem_b, sem_c):
        cid = lax.axis_index("c")
        sid = lax.axis_index("s")
        wid = sid * 2 + cid
        zv = jnp.zeros((16,), jnp.float32)

        def zbody(r, _):
            for j in range(HID // 16):
                rows_v[r, pl.ds(j * 16, 16)] = zv
            w_v[r, :] = zv
            return 0

        lax.fori_loop(0, K, zbody, 0)
        row0 = sid * TROWS
        for t in range(TROWS // ZR):
            pltpu.sync_copy(rows_v, acc_s.at[pl.ds(row0 + t * ZR, ZR)])
            pltpu.sync_copy(w_v, den_s.at[pl.ds(row0 + t * ZR, ZR)])
        plsc.subcore_barrier()

        gdn = lax.GatherDimensionNumbers(
            offset_dims=(), collapsed_slice_dims=(0,), start_index_map=(0,))

        def chunk(t, _):
            ch = t * NW + wid
            la = pltpu.async_copy(ae_hbm.at[pl.ds(ch * (K // 8), K // 8)],
                                  ae_v, sem_c)
            pltpu.sync_copy(rec_hbm.at[ch], rec_v)
            g0 = pltpu.async_copy(asrc_hbm.at[rec_v.at[0]], asrc_v, sem_a)
            g1 = pltpu.async_copy(adst_hbm.at[rec_v.at[1]], adst_v, sem_b)
            g2 = pltpu.async_copy(xw_hbm.at[rec_v.at[0]], rows_v, sem_c)
            la.wait()
            g0.wait()
            g1.wait()

            def wbody(g, _):
                for p in range(8):
                    i = g * 8 + p
                    a = (asrc_v[i, :] + adst_v[i, :]
                         + ae_v[g, pl.ds(p * 16, 16)])
                    a = jnp.where(a >= 0.0, a, 0.2 * a)
                    w_v[i, :] = jnp.exp(a)
                return 0

            lax.fori_loop(0, K // 8, wbody, 0)
            sd = pltpu.async_copy(w_v, den_s.at[rec_v.at[1]], sem_a, add=True)
            g2.wait()

            def mbody(e, _):
                wrow = w_v[e, :]
                for j in range(HID // 16):
                    hidx = jnp.full((16, 1), j // 2, jnp.int32)
                    wb = lax.gather(
                        wrow, hidx, gdn, (1,),
                        mode=lax.GatherScatterMode.PROMISE_IN_BOUNDS)
                    sl = pl.ds(j * 16, 16)
                    rows_v[e, sl] = rows_v[e, sl] * wb
                return 0

            lax.fori_loop(0, K, mbody, 0)
            sd.wait()
            pltpu.sync_copy(rows_v, acc_s.at[rec_v.at[1]], add=True)
            return 0

        lax.fori_loop(0, TPW, chunk, 0)
        plsc.subcore_barrier()

        for t in range(TROWS // ZR):
            r = row0 + t * ZR
            pltpu.sync_copy(acc_s.at[pl.ds(r, ZR)], rows_v)
            pltpu.sync_copy(rows_v, out_hbm.at[cid, pl.ds(r, ZR)])
            pltpu.sync_copy(den_s.at[pl.ds(r, ZR)], w_v)
            pltpu.sync_copy(w_v, den_hbm.at[cid, pl.ds(r, ZR)])

    return k(rec, ae, asrc, adst, xw)


def _post(outp, denp, bgat2, W_ih, bias2, E4, gamma2, beta2):
    BN = 1000

    def body(op_ref, dp_ref, bg_ref, wih_ref, b2_ref, e4_ref, ga_ref, be_ref,
             hout_ref, h_ref, c_ref):
        s = op_ref[0] + op_ref[1]
        den = dp_ref[0] + dp_ref[1]
        denf = jnp.dot(den, e4_ref[...], preferred_element_type=jnp.float32)
        sf = s / (denf + 1e-16) + bg_ref[...]
        gates = lax.dot_general(sf, wih_ref[...],
                                (((1,), (1,)), ((), ())),
                                preferred_element_type=jnp.float32) + b2_ref[...]
        ig = jax.nn.sigmoid(gates[:, 0:HID])
        gg = jnp.tanh(gates[:, 2 * HID:3 * HID])
        og = jax.nn.sigmoid(gates[:, 3 * HID:4 * HID])
        c = ig * gg
        h = og * jnp.tanh(c)
        mu = jnp.mean(h, axis=1, keepdims=True)
        var = jnp.mean((h - mu) * (h - mu), axis=1, keepdims=True)
        hn = (h - mu) / jnp.sqrt(var + 1e-5) * ga_ref[...] + be_ref[...]
        hout_ref[...] = hn
        h_ref[...] = h
        c_ref[...] = c

    return pl.pallas_call(
        body,
        grid=(N // BN,),
        in_specs=[
            pl.BlockSpec((2, BN, HID), lambda i: (0, i, 0)),
            pl.BlockSpec((2, BN, HP), lambda i: (0, i, 0)),
            pl.BlockSpec((1, HID), lambda i: (0, 0)),
            pl.BlockSpec((4 * HID, HID), lambda i: (0, 0)),
            pl.BlockSpec((1, 4 * HID), lambda i: (0, 0)),
            pl.BlockSpec((HP, HID), lambda i: (0, 0)),
            pl.BlockSpec((1, HID), lambda i: (0, 0)),
            pl.BlockSpec((1, HID), lambda i: (0, 0)),
        ],
        out_specs=[
            pl.BlockSpec((BN, HID), lambda i: (i, 0)),
            pl.BlockSpec((BN, HID), lambda i: (i, 0)),
            pl.BlockSpec((BN, HID), lambda i: (i, 0)),
        ],
        out_shape=[
            jax.ShapeDtypeStruct((N, HID), jnp.float32),
            jax.ShapeDtypeStruct((N, HID), jnp.float32),
            jax.ShapeDtypeStruct((N, HID), jnp.float32),
        ],
    )(outp, denp, bgat2, W_ih, bias2, E4, gamma2, beta2)


def kernel(x, edge_index, edge_attr, W, att_src, att_dst, att_edge, W_edge,
           b_gat, W_ih, W_hh, b_ih, b_hh, gamma, beta):
    x2d = x.reshape(N, FIN)
    pad_e = EPAD - E
    src = jnp.concatenate([edge_index[0], jnp.zeros((pad_e,), jnp.int32)])
    dst = jnp.concatenate([edge_index[1], jnp.full((pad_e,), N, jnp.int32)])

    hmask = (jnp.arange(HP)[None, :] == jnp.arange(H)[:, None]).astype(jnp.float32)
    As = (att_src[:, :, None] * hmask[:, None, :]).reshape(FIN, HP)
    Ad = (att_dst[:, :, None] * hmask[:, None, :]).reshape(FIN, HP)
    Vp = ((W_edge.reshape(ED, H, C) * att_edge[None]).sum(-1) @ hmask)
    BD = jnp.kron(jnp.eye(8, dtype=jnp.float32), Vp)
    E4 = (jnp.arange(HP)[:, None] == (jnp.arange(HID)[None, :] // C)).astype(jnp.float32)
    bias2 = (b_ih + b_hh).reshape(1, 4 * HID)
    bgat2 = b_gat.reshape(1, HID)
    gamma2 = gamma.reshape(1, HID)
    beta2 = beta.reshape(1, HID)

    xw, asrc, adst = _prep_nodes(x2d, W, As, Ad)
    zpad = jnp.zeros((NP - N, HP), jnp.float32)
    asrc = jnp.concatenate([asrc, zpad])
    adst = jnp.concatenate([adst, zpad])
    ae = _prep_edges(edge_attr.reshape(E // 8, 128), BD)
    nch = EPAD // K
    rec = jnp.stack([src.reshape(nch, K), dst.reshape(nch, K)], axis=1)
    outp, denp = _sc_gat(rec, ae, asrc, adst, xw)
    hout, h, c = _post(outp, denp, bgat2, W_ih, bias2, E4, gamma2, beta2)
    return (hout.reshape(1, N, HID), h.reshape(1, N, HID), c.reshape(1, N, HID))


# speedup vs baseline: 1.7417x; 1.5182x over previous
import functools

import jax
import jax.numpy as jnp
from jax import lax
from jax.experimental import pallas as pl
from jax.experimental.pallas import tpu as pltpu
from jax.experimental.pallas import tpu_sc as plsc

N = 10000
E = 320000
H = 4
C = 32
HID = 128
FIN = 128
ED = 16
HP = 16
K = 128
NW = 32
NCH = E // K
TPW = -(-NCH // NW)
NP = 10240
TROWS = NP // 16
ZR = 128


def _prep_nodes(x2d, W, As, Ad):
    BN = 1000

    def body(x_ref, w_ref, as_ref, ad_ref, xw_ref, als_ref, ald_ref):
        xw = jnp.dot(x_ref[...], w_ref[...], preferred_element_type=jnp.float32)
        xw_ref[...] = xw
        als_ref[...] = jnp.dot(xw, as_ref[...], preferred_element_type=jnp.float32)
        ald_ref[...] = jnp.dot(xw, ad_ref[...], preferred_element_type=jnp.float32)

    return pl.pallas_call(
        body,
        grid=(N // BN,),
        in_specs=[
            pl.BlockSpec((BN, FIN), lambda i: (i, 0)),
            pl.BlockSpec((FIN, HID), lambda i: (0, 0)),
            pl.BlockSpec((HID, HP), lambda i: (0, 0)),
            pl.BlockSpec((HID, HP), lambda i: (0, 0)),
        ],
        out_specs=[
            pl.BlockSpec((BN, HID), lambda i: (i, 0)),
            pl.BlockSpec((BN, HP), lambda i: (i, 0)),
            pl.BlockSpec((BN, HP), lambda i: (i, 0)),
        ],
        out_shape=[
            jax.ShapeDtypeStruct((N, HID), jnp.float32),
            jax.ShapeDtypeStruct((N, HP), jnp.float32),
            jax.ShapeDtypeStruct((N, HP), jnp.float32),
        ],
    )(x2d, W, As, Ad)


def _prep_edges(ea_pack, BD):
    BE = 4000

    def body(ea_ref, bd_ref, ae_ref):
        ae_ref[...] = jnp.dot(ea_ref[...], bd_ref[...],
                              preferred_element_type=jnp.float32)

    return pl.pallas_call(
        body,
        grid=(E // 8 // BE,),
        in_specs=[
            pl.BlockSpec((BE, 128), lambda i: (i, 0)),
            pl.BlockSpec((128, 128), lambda i: (0, 0)),
        ],
        out_specs=pl.BlockSpec((BE, 128), lambda i: (i, 0)),
        out_shape=jax.ShapeDtypeStruct((E // 8, 128), jnp.float32),
    )(ea_pack, BD)


def _sc_gat(src, dst, ae, asrc, adst, xw):
    mesh = plsc.VectorSubcoreMesh(core_axis_name="c", subcore_axis_name="s")

    @functools.partial(
        pl.kernel,
        out_type=[
            jax.ShapeDtypeStruct((2, NP, HID), jnp.float32),
            jax.ShapeDtypeStruct((2, NP, HP), jnp.float32),
        ],
        mesh=mesh,
        compiler_params=pltpu.CompilerParams(use_tc_tiling_on_sc=False),
        scratch_types=[
            pltpu.VMEM_SHARED((NP, HID), jnp.float32),
            pltpu.VMEM_SHARED((NP, HP), jnp.float32),
            pltpu.VMEM((K,), jnp.int32),
            pltpu.VMEM((K,), jnp.int32),
            pltpu.VMEM((K, HP), jnp.float32),
            pltpu.VMEM((K, HP), jnp.float32),
            pltpu.VMEM((K // 8, HID), jnp.float32),
            pltpu.VMEM((K, HP), jnp.float32),
            pltpu.VMEM((K, HID), jnp.float32),
            pltpu.SemaphoreType.DMA,
            pltpu.SemaphoreType.DMA,
            pltpu.SemaphoreType.DMA,
        ],
    )
    def k(src_hbm, dst_hbm, ae_hbm, asrc_hbm, adst_hbm, xw_hbm,
          out_hbm, den_hbm,
          acc_s, den_s, idx_s, idx_d, asrc_v, adst_v, ae_v, w_v, rows_v,
          sem_a, sem_b, sem_c):
        cid = lax.axis_index("c")
        sid = lax.axis_index("s")
        wid = sid * 2 + cid
        zv = jnp.zeros((16,), jnp.float32)

        def zbody(r, _):
            for j in range(HID // 16):
                rows_v[r, pl.ds(j * 16, 16)] = zv
            w_v[r, :] = zv
            return 0

        lax.fori_loop(0, K, zbody, 0)
        row0 = sid * TROWS
        for t in range(TROWS // ZR):
            pltpu.sync_copy(rows_v.at[pl.ds(0, ZR)],
                            acc_s.at[pl.ds(row0 + t * ZR, ZR)])
            pltpu.sync_copy(w_v.at[pl.ds(0, ZR)],
                            den_s.at[pl.ds(row0 + t * ZR, ZR)])
        plsc.subcore_barrier()

        def chunk(t, _):
            ch = t * NW + wid

            @pl.when(ch < NCH)
            def _():
                base = ch * K
                cp0 = pltpu.async_copy(src_hbm.at[pl.ds(base, K)], idx_s, sem_a)
                cp1 = pltpu.async_copy(dst_hbm.at[pl.ds(base, K)], idx_d, sem_b)
                cp2 = pltpu.async_copy(ae_hbm.at[pl.ds(ch * (K // 8), K // 8)],
                                       ae_v, sem_c)
                cp0.wait()
                cp1.wait()
                g0 = pltpu.async_copy(asrc_hbm.at[idx_s], asrc_v, sem_a)
                g1 = pltpu.async_copy(adst_hbm.at[idx_d], adst_v, sem_b)
                g2 = pltpu.async_copy(xw_hbm.at[idx_s], rows_v, sem_a)
                cp2.wait()
                g0.wait()
                g1.wait()

                def wbody(g, _):
                    for p in range(8):
                        i = g * 8 + p
                        a = (asrc_v[i, :] + adst_v[i, :]
                             + ae_v[g, pl.ds(p * 16, 16)])
                        a = jnp.where(a >= 0.0, a, 0.2 * a)
                        w_v[i, :] = jnp.exp(a)
                    return 0

                lax.fori_loop(0, K // 8, wbody, 0)
                pltpu.sync_copy(w_v, den_s.at[idx_d], add=True)
                g2.wait()

                gdn = lax.GatherDimensionNumbers(
                    offset_dims=(), collapsed_slice_dims=(0,),
                    start_index_map=(0,))

                def mbody(e, _):
                    wrow = w_v[e, :]
                    for j in range(HID // 16):
                        hidx = jnp.full((16, 1), j // 2, jnp.int32)
                        wb = lax.gather(
                            wrow, hidx, gdn, (1,),
                            mode=lax.GatherScatterMode.PROMISE_IN_BOUNDS)
                        sl = pl.ds(j * 16, 16)
                        rows_v[e, sl] = rows_v[e, sl] * wb
                    return 0

                lax.fori_loop(0, K, mbody, 0)
                pltpu.sync_copy(rows_v, acc_s.at[idx_d], add=True)

            return 0

        lax.fori_loop(0, TPW, chunk, 0)
        plsc.subcore_barrier()

        for t in range(TROWS // ZR):
            r = row0 + t * ZR
            pltpu.sync_copy(acc_s.at[pl.ds(r, ZR)], rows_v.at[pl.ds(0, ZR)])
            pltpu.sync_copy(rows_v.at[pl.ds(0, ZR)],
                            out_hbm.at[cid, pl.ds(r, ZR)])
            pltpu.sync_copy(den_s.at[pl.ds(r, ZR)], w_v.at[pl.ds(0, ZR)])
            pltpu.sync_copy(w_v.at[pl.ds(0, ZR)],
                            den_hbm.at[cid, pl.ds(r, ZR)])

    return k(src, dst, ae, asrc, adst, xw)


def _post(outp, denp, bgat2, W_ih, bias2, E4, gamma2, beta2):
    BN = 1000

    def body(op_ref, dp_ref, bg_ref, wih_ref, b2_ref, e4_ref, ga_ref, be_ref,
             hout_ref, h_ref, c_ref):
        s = op_ref[0] + op_ref[1]
        den = dp_ref[0] + dp_ref[1]
        denf = jnp.dot(den, e4_ref[...], preferred_element_type=jnp.float32)
        sf = s / (denf + 1e-16) + bg_ref[...]
        gates = lax.dot_general(sf, wih_ref[...],
                                (((1,), (1,)), ((), ())),
                                preferred_element_type=jnp.float32) + b2_ref[...]
        ig = jax.nn.sigmoid(gates[:, 0:HID])
        gg = jnp.tanh(gates[:, 2 * HID:3 * HID])
        og = jax.nn.sigmoid(gates[:, 3 * HID:4 * HID])
        c = ig * gg
        h = og * jnp.tanh(c)
        mu = jnp.mean(h, axis=1, keepdims=True)
        var = jnp.mean((h - mu) * (h - mu), axis=1, keepdims=True)
        hn = (h - mu) / jnp.sqrt(var + 1e-5) * ga_ref[...] + be_ref[...]
        hout_ref[...] = hn
        h_ref[...] = h
        c_ref[...] = c

    return pl.pallas_call(
        body,
        grid=(N // BN,),
        in_specs=[
            pl.BlockSpec((2, BN, HID), lambda i: (0, i, 0)),
            pl.BlockSpec((2, BN, HP), lambda i: (0, i, 0)),
            pl.BlockSpec((1, HID), lambda i: (0, 0)),
            pl.BlockSpec((4 * HID, HID), lambda i: (0, 0)),
            pl.BlockSpec((1, 4 * HID), lambda i: (0, 0)),
            pl.BlockSpec((HP, HID), lambda i: (0, 0)),
            pl.BlockSpec((1, HID), lambda i: (0, 0)),
            pl.BlockSpec((1, HID), lambda i: (0, 0)),
        ],
        out_specs=[
            pl.BlockSpec((BN, HID), lambda i: (i, 0)),
            pl.BlockSpec((BN, HID), lambda i: (i, 0)),
            pl.BlockSpec((BN, HID), lambda i: (i, 0)),
        ],
        out_shape=[
            jax.ShapeDtypeStruct((N, HID), jnp.float32),
            jax.ShapeDtypeStruct((N, HID), jnp.float32),
            jax.ShapeDtypeStruct((N, HID), jnp.float32),
        ],
    )(outp, denp, bgat2, W_ih, bias2, E4, gamma2, beta2)


def kernel(x, edge_index, edge_attr, W, att_src, att_dst, att_edge, W_edge,
           b_gat, W_ih, W_hh, b_ih, b_hh, gamma, beta):
    x2d = x.reshape(N, FIN)
    src = edge_index[0]
    dst = edge_index[1]

    hmask = (jnp.arange(HP)[None, :] == jnp.arange(H)[:, None]).astype(jnp.float32)
    As = (att_src[:, :, None] * hmask[:, None, :]).reshape(FIN, HP)
    Ad = (att_dst[:, :, None] * hmask[:, None, :]).reshape(FIN, HP)
    Vp = ((W_edge.reshape(ED, H, C) * att_edge[None]).sum(-1) @ hmask)
    BD = jnp.kron(jnp.eye(8, dtype=jnp.float32), Vp)
    E4 = (jnp.arange(HP)[:, None] == (jnp.arange(HID)[None, :] // C)).astype(jnp.float32)
    bias2 = (b_ih + b_hh).reshape(1, 4 * HID)
    bgat2 = b_gat.reshape(1, HID)
    gamma2 = gamma.reshape(1, HID)
    beta2 = beta.reshape(1, HID)

    xw, asrc, adst = _prep_nodes(x2d, W, As, Ad)
    ae = _prep_edges(edge_attr.reshape(E // 8, 128), BD)
    outp, denp = _sc_gat(src, dst, ae, asrc, adst, xw)
    hout, h, c = _post(outp, denp, bgat2, W_ih, bias2, E4, gamma2, beta2)
    return (hout.reshape(1, N, HID), h.reshape(1, N, HID), c.reshape(1, N, HID))
